# Initial kernel scaffold; baseline (speedup 1.0000x reference)
import jax
import jax.numpy as jnp
from jax.experimental import pallas as pl


def kernel(x, edge_index, W1, b1, W2, b2):
    raise NotImplementedError("write your pallas kernel here")


# speedup vs baseline: 10.3339x; 10.3339x over previous
import functools

import jax
import jax.numpy as jnp
from jax import lax
from jax.experimental import pallas as pl
from jax.experimental.pallas import tpu as pltpu
from jax.experimental.pallas import tpu_sc as plsc

NNODE = 10000
DMODEL = 128
NCORE = 2
NSUB = 16
NTILE = NCORE * NSUB
CHUNK = 128
NPAD = 10112
RPT = NPAD // NSUB
DEGW = 128

_mesh = plsc.VectorSubcoreMesh(core_axis_name="c", subcore_axis_name="s")


def _deg_kernel(epad):
    nch = epad // (NTILE * CHUNK)
    ept = nch * CHUNK

    @functools.partial(
        pl.kernel,
        out_type=jax.ShapeDtypeStruct((NCORE, NPAD, DEGW), jnp.float32),
        mesh=_mesh,
        scratch_types=[
            pltpu.VMEM((CHUNK,), jnp.int32),
            pltpu.VMEM((CHUNK, DEGW), jnp.float32),
            pltpu.VMEM_SHARED((NPAD, DEGW), jnp.float32),
        ],
    )
    def k(dst_hbm, ones_hbm, zeros_hbm, out_hbm, dstv, onesv, acc):
        c = lax.axis_index("c")
        s = lax.axis_index("s")
        wid = c * NSUB + s
        r0 = s * RPT
        pltpu.sync_copy(zeros_hbm.at[pl.ds(r0, RPT)], acc.at[pl.ds(r0, RPT)])
        pltpu.sync_copy(ones_hbm, onesv)
        plsc.subcore_barrier()
        e0 = wid * ept

        def body(i, carry):
            base = pl.multiple_of(e0 + i * CHUNK, CHUNK)
            pltpu.sync_copy(dst_hbm.at[pl.ds(base, CHUNK)], dstv)
            pltpu.sync_copy(onesv, acc.at[dstv], add=True)
            return carry

        lax.fori_loop(0, nch, body, 0)
        plsc.subcore_barrier()
        pltpu.sync_copy(acc.at[pl.ds(r0, RPT)], out_hbm.at[c, pl.ds(r0, RPT)])

    return k


def _agg_kernel(epad):
    nch = epad // (NTILE * CHUNK)
    ept = nch * CHUNK

    @functools.partial(
        pl.kernel,
        out_type=jax.ShapeDtypeStruct((NCORE, NPAD, DMODEL), jnp.float32),
        mesh=_mesh,
        scratch_types=[
            pltpu.VMEM((CHUNK,), jnp.int32),
            pltpu.VMEM((CHUNK,), jnp.int32),
            pltpu.VMEM((CHUNK, DMODEL), jnp.float32),
            pltpu.VMEM_SHARED((NPAD, DMODEL), jnp.float32),
            pltpu.SemaphoreType.DMA,
        ],
    )
    def k(g_hbm, src_hbm, dst_hbm, zeros_hbm, out_hbm, srcv, dstv, rows, acc, sem):
        c = lax.axis_index("c")
        s = lax.axis_index("s")
        wid = c * NSUB + s
        r0 = s * RPT
        pltpu.sync_copy(zeros_hbm.at[pl.ds(r0, RPT)], acc.at[pl.ds(r0, RPT)])
        plsc.subcore_barrier()
        e0 = wid * ept

        def body(i, carry):
            base = pl.multiple_of(e0 + i * CHUNK, CHUNK)
            pltpu.sync_copy(src_hbm.at[pl.ds(base, CHUNK)], srcv)
            pltpu.sync_copy(dst_hbm.at[pl.ds(base, CHUNK)], dstv)
            pltpu.async_copy(g_hbm.at[srcv], rows, sem).wait()
            pltpu.sync_copy(rows, acc.at[dstv], add=True)
            return carry

        lax.fori_loop(0, nch, body, 0)
        plsc.subcore_barrier()
        pltpu.sync_copy(acc.at[pl.ds(r0, RPT)], out_hbm.at[c, pl.ds(r0, RPT)])

    return k


def _tc_a(degp_ref, x_ref, w1_ref, dis_ref, g1_ref):
    deg = degp_ref[0, 0:NNODE, 0:1] + degp_ref[1, 0:NNODE, 0:1] + 1.0
    dis = lax.rsqrt(deg)
    dis_ref[...] = dis
    h = jnp.dot(x_ref[...], w1_ref[...], preferred_element_type=jnp.float32)
    g1_ref[...] = dis * h


def _tc_b(p_ref, g_ref, dis_ref, b1_ref, w2_ref, g2_ref):
    s = p_ref[0, 0:NNODE, :] + p_ref[1, 0:NNODE, :]
    dis = dis_ref[...]
    z = jnp.maximum(dis * (s + g_ref[...]) + b1_ref[...], 0.0)
    h2 = jnp.dot(z, w2_ref[...], preferred_element_type=jnp.float32)
    g2_ref[...] = dis * h2


def _tc_c(p_ref, g_ref, dis_ref, b2_ref, out_ref):
    s = p_ref[0, 0:NNODE, :] + p_ref[1, 0:NNODE, :]
    out_ref[...] = dis_ref[...] * (s + g_ref[...]) + b2_ref[...]


def kernel(x, edge_index, W1, b1, W2, b2):
    e = edge_index.shape[1]
    grain = NTILE * CHUNK
    epad = ((e + grain - 1) // grain) * grain
    src = edge_index[0].astype(jnp.int32)
    dst = edge_index[1].astype(jnp.int32)
    pad = epad - e
    src = jnp.concatenate([src, jnp.zeros((pad,), jnp.int32)])
    dst = jnp.concatenate([dst, jnp.full((pad,), NNODE, jnp.int32)])

    ones = jnp.ones((CHUNK, DEGW), jnp.float32)
    zeros_deg = jnp.zeros((NPAD, DEGW), jnp.float32)
    zeros_acc = jnp.zeros((NPAD, DMODEL), jnp.float32)

    degp = _deg_kernel(epad)(dst, ones, zeros_deg)

    dis, g1 = pl.pallas_call(
        _tc_a,
        out_shape=(
            jax.ShapeDtypeStruct((NNODE, 1), jnp.float32),
            jax.ShapeDtypeStruct((NNODE, DMODEL), jnp.float32),
        ),
    )(degp, x, W1)

    agg = _agg_kernel(epad)
    p1 = agg(g1, src, dst, zeros_acc)

    g2 = pl.pallas_call(
        _tc_b,
        out_shape=jax.ShapeDtypeStruct((NNODE, DMODEL), jnp.float32),
    )(p1, g1, dis, b1.reshape(1, DMODEL), W2)

    p2 = agg(g2, src, dst, zeros_acc)

    out = pl.pallas_call(
        _tc_c,
        out_shape=jax.ShapeDtypeStruct((NNODE, DMODEL), jnp.float32),
    )(p2, g2, dis, b2.reshape(1, DMODEL))

    return out
